# baseline (speedup 1.0000x reference)
import jax
import jax.numpy as jnp
from jax.experimental import pallas as pl

N = 8192
C = 8
K = 512
R = 256


def _select_kernel(x_ref, k_ref, o_ref):
    x = x_ref[...]
    kk = k_ref[...]
    b = jax.lax.bitcast_convert_type(x, jnp.int32)
    key = jnp.where(b < 0, (~b) ^ jnp.int32(-2**31), b)

    kf = kk.astype(jnp.float32)
    lo0 = jnp.full((R, 1), -2**31, jnp.int32)
    hi0 = jnp.full((R, 1), 2**31 - 1, jnp.int32)
    t0 = jnp.full((R, 1), 2**31 - 1, jnp.int32)
    found0 = jnp.zeros((R, 1), jnp.int32)

    def cond(carry):
        it, done, lo, hi, tt, found = carry
        return jnp.logical_and(it < 34, done == 0)

    def body(carry):
        it, done, lo, hi, tt, found = carry
        notf = found == 0
        collapsed = jnp.logical_and(lo >= hi, notf)
        tt = jnp.where(collapsed, lo, tt)
        found = jnp.where(collapsed, 1, found)
        mid = (lo >> 1) + (hi >> 1) + (lo & hi & 1)
        gt = jnp.where(key > mid, 1.0, 0.0)
        cnt = jnp.sum(gt, axis=1, keepdims=True)
        hit = jnp.logical_and(cnt == kf, jnp.logical_and(notf, jnp.logical_not(collapsed)))
        tt = jnp.where(hit, mid, tt)
        found = jnp.where(hit, 1, found)
        go_up = cnt > kf
        lo = jnp.where(go_up, mid + 1, lo)
        hi = jnp.where(go_up, hi, mid)
        done = jnp.min(found, axis=None)
        return it + 1, done, lo, hi, tt, found

    carry = (jnp.int32(0), jnp.int32(0), lo0, hi0, t0, found0)
    _, _, _, _, tt, _ = jax.lax.while_loop(cond, body, carry)

    a = jnp.max(jnp.where(key <= tt, key, -2**31), axis=1, keepdims=True)
    eq = key == a
    m = jnp.sum(jnp.where(key > a, 1.0, 0.0), axis=1, keepdims=True)
    ii = jax.lax.broadcasted_iota(jnp.int32, (K, K), 0)
    jj = jax.lax.broadcasted_iota(jnp.int32, (K, K), 1)
    tri = jnp.where(ii < jj, 1.0, 0.0).astype(jnp.bfloat16)
    eqf = jnp.where(eq, 1.0, 0.0).astype(jnp.bfloat16)
    t = jax.lax.dot(eqf, tri, preferred_element_type=jnp.float32)
    sel = jnp.logical_and(eq, t == (kf - m))
    o_ref[...] = jnp.where(sel, 1.0, 0.0).astype(jnp.float32)


def kernel(latents, k):
    x = latents.reshape(N * C, K)
    k_rows = jnp.tile(k.astype(jnp.int32), N).reshape(N * C, 1)
    out = pl.pallas_call(
        _select_kernel,
        grid=(N * C // R,),
        in_specs=[
            pl.BlockSpec((R, K), lambda i: (i, 0)),
            pl.BlockSpec((R, 1), lambda i: (i, 0)),
        ],
        out_specs=pl.BlockSpec((R, K), lambda i: (i, 0)),
        out_shape=jax.ShapeDtypeStruct((N * C, K), jnp.float32),
    )(x, k_rows)
    return out.reshape(N, C * K)


# speedup vs baseline: 1.4217x; 1.4217x over previous
import jax
import jax.numpy as jnp
from jax.experimental import pallas as pl

N = 8192
C = 8
K = 512
R = 256


def _select_kernel(x_ref, k_ref, o_ref):
    x = x_ref[...]
    kk = k_ref[...]
    b = jax.lax.bitcast_convert_type(x, jnp.int32)
    key = jnp.where(b < 0, (~b) ^ jnp.int32(-2**31), b)

    kf = kk.astype(jnp.float32)
    lo0 = jnp.full((R, 1), -2**31, jnp.int32)
    hi0 = jnp.full((R, 1), 2**31 - 1, jnp.int32)
    def body(_, carry):
        lo, hi = carry
        mid = (lo >> 1) + (hi >> 1) + (lo & hi & 1)
        gt = jnp.where(key > mid, 1.0, 0.0)
        cnt = jnp.sum(gt, axis=1, keepdims=True)
        go_up = cnt > kf
        lo = jnp.where(go_up, mid + 1, lo)
        hi = jnp.where(go_up, hi, mid)
        return lo, hi

    lo, _ = jax.lax.fori_loop(0, 32, body, (lo0, hi0))
    a = lo
    eq = key == a
    m = jnp.sum(jnp.where(key > a, 1.0, 0.0), axis=1, keepdims=True)
    ii = jax.lax.broadcasted_iota(jnp.int32, (K, K), 0)
    jj = jax.lax.broadcasted_iota(jnp.int32, (K, K), 1)
    tri = jnp.where(ii < jj, 1.0, 0.0).astype(jnp.bfloat16)
    eqf = jnp.where(eq, 1.0, 0.0).astype(jnp.bfloat16)
    t = jax.lax.dot(eqf, tri, preferred_element_type=jnp.float32)
    sel = jnp.logical_and(eq, t == (kf - m))
    o_ref[...] = jnp.where(sel, 1.0, 0.0).astype(jnp.float32)


def kernel(latents, k):
    x = latents.reshape(N * C, K)
    k_rows = jnp.tile(k.astype(jnp.int32), N).reshape(N * C, 1)
    out = pl.pallas_call(
        _select_kernel,
        grid=(N * C // R,),
        in_specs=[
            pl.BlockSpec((R, K), lambda i: (i, 0)),
            pl.BlockSpec((R, 1), lambda i: (i, 0)),
        ],
        out_specs=pl.BlockSpec((R, K), lambda i: (i, 0)),
        out_shape=jax.ShapeDtypeStruct((N * C, K), jnp.float32),
    )(x, k_rows)
    return out.reshape(N, C * K)


# speedup vs baseline: 3.3769x; 2.3752x over previous
import jax
import jax.numpy as jnp
from jax.experimental import pallas as pl

N = 8192
C = 8
K = 512
R = 1024


def _select_kernel(x_ref, k_ref, o_ref):
    x = x_ref[...]
    kk = k_ref[0]
    xt = x.T
    b = jax.lax.bitcast_convert_type(xt, jnp.int32)
    key = jnp.where(b < 0, (~b) ^ jnp.int32(-2**31), b)

    kf = kk.astype(jnp.float32)
    lo0 = jnp.full((1, R), -2**31, jnp.int32)
    hi0 = jnp.full((1, R), 2**31 - 1, jnp.int32)

    def body(_, carry):
        lo, hi = carry
        mid = (lo >> 1) + (hi >> 1) + (lo & hi & 1)
        gt = jnp.where(key > mid, 1.0, 0.0)
        cnt = jnp.sum(gt, axis=0, keepdims=True)
        go_up = cnt > kf
        lo = jnp.where(go_up, mid + 1, lo)
        hi = jnp.where(go_up, hi, mid)
        return lo, hi

    lo, _ = jax.lax.fori_loop(0, 32, body, (lo0, hi0))
    a = lo
    eq = key == a
    m = jnp.sum(jnp.where(key > a, 1.0, 0.0), axis=0, keepdims=True)
    ii = jax.lax.broadcasted_iota(jnp.int32, (K, K), 0)
    jj = jax.lax.broadcasted_iota(jnp.int32, (K, K), 1)
    tril = jnp.where(jj < ii, 1.0, 0.0).astype(jnp.bfloat16)
    eqf = jnp.where(eq, 1.0, 0.0).astype(jnp.bfloat16)
    t = jax.lax.dot(tril, eqf, preferred_element_type=jnp.float32)
    sel = jnp.logical_and(eq, t == (kf - m))
    o_ref[...] = jnp.where(sel, 1.0, 0.0).astype(jnp.float32).T


def kernel(latents, k):
    x = latents.reshape(N * C, K)
    k_rows = jnp.tile(k.astype(jnp.int32), N).reshape(N * C // R, 1, R)
    out = pl.pallas_call(
        _select_kernel,
        grid=(N * C // R,),
        in_specs=[
            pl.BlockSpec((R, K), lambda i: (i, 0)),
            pl.BlockSpec((1, 1, R), lambda i: (i, 0, 0)),
        ],
        out_specs=pl.BlockSpec((R, K), lambda i: (i, 0)),
        out_shape=jax.ShapeDtypeStruct((N * C, K), jnp.float32),
    )(x, k_rows)
    return out.reshape(N, C * K)
